# baseline (speedup 1.0000x reference)
import functools

import jax
import jax.numpy as jnp
from jax.experimental import pallas as pl
from jax.experimental.pallas import tpu as pltpu


def _round_up(x, m):
    return ((x + m - 1) // m) * m


def _ae_loss_kernel(x_ref, w1, b1, w23, b23, w4, b4, out_ref, *,
                    m_valid, tile_m):
    x = x_ref[...]
    xb = x.astype(jnp.bfloat16)
    h = jnp.tanh(
        jnp.dot(xb, w1[...], preferred_element_type=jnp.float32) + b1[...])
    h2 = jnp.tanh(
        jnp.dot(h.astype(jnp.bfloat16), w23[...],
                preferred_element_type=jnp.float32) + b23[...])
    dec = (jnp.dot(h2.astype(jnp.bfloat16), w4[...],
                   preferred_element_type=jnp.float32) + b4[...])
    d = x - dec
    if m_valid is not None:
        i = pl.program_id(0)
        rows = i * tile_m + jax.lax.broadcasted_iota(jnp.int32, (tile_m, 1), 0)
        d = jnp.where(rows < m_valid, d, 0.0)
    out_ref[0] = jnp.sum(d * d, axis=0, keepdims=True)


def _resident(arr):
    nd = arr.ndim
    return pl.BlockSpec(arr.shape, lambda *_: (0,) * nd)


TILE_M = 1024


def kernel(grid_feature, w1p, b1p, w2p, b2p, w3p, b3p, w4p, b4p):
    F = w1p.shape[0]
    x = jnp.asarray(grid_feature, jnp.float32).reshape(-1, F)
    m = x.shape[0]
    tile_m = min(TILE_M, _round_up(m, 8))
    m_pad = _round_up(m, tile_m)
    n_tiles = m_pad // tile_m
    if m_pad != m:
        x = jnp.pad(x, ((0, m_pad - m), (0, 0)))
    m_valid = None if m_pad == m else m

    w23 = w2p @ w3p
    b23 = b2p @ w3p + b3p

    MID = w1p.shape[1]
    weights = (w1p.astype(jnp.bfloat16), b1p,
               w23.astype(jnp.bfloat16), b23,
               w4p.astype(jnp.bfloat16), b4p)

    flops = 2 * m_pad * (F * MID + MID * MID + MID * F) + 4 * m_pad * F
    trans = 2 * m_pad * MID
    bytes_acc = 4 * m_pad * F + 2 * (2 * F * MID + MID * MID) + 4 * n_tiles * F

    partials = pl.pallas_call(
        functools.partial(_ae_loss_kernel, m_valid=m_valid, tile_m=tile_m),
        out_shape=jax.ShapeDtypeStruct((n_tiles, 1, F), jnp.float32),
        grid_spec=pltpu.PrefetchScalarGridSpec(
            num_scalar_prefetch=0,
            grid=(n_tiles,),
            in_specs=[pl.BlockSpec((tile_m, F), lambda i: (i, 0))]
                     + [_resident(w) for w in weights],
            out_specs=pl.BlockSpec((1, 1, F), lambda i: (i, 0, 0)),
        ),
        compiler_params=pltpu.CompilerParams(
            dimension_semantics=("parallel",),
            vmem_limit_bytes=64 * 1024 * 1024),
        cost_estimate=pl.CostEstimate(
            flops=flops, transcendentals=trans, bytes_accessed=bytes_acc),
    )(x, *weights)
    return jnp.sum(partials)


# speedup vs baseline: 1.4857x; 1.4857x over previous
import functools

import jax
import jax.numpy as jnp
from jax.experimental import pallas as pl
from jax.experimental.pallas import tpu as pltpu


def _round_up(x, m):
    return ((x + m - 1) // m) * m


def _ae_loss_kernel(x_ref, w1, b1, w23, b23, w4, b4, out_ref, *,
                    m_valid, tile_m):
    x = x_ref[...]
    xb = x.astype(jnp.bfloat16)
    h = jnp.tanh(
        jnp.dot(xb, w1[...], preferred_element_type=jnp.float32) + b1[...])
    h2 = jnp.tanh(
        jnp.dot(h.astype(jnp.bfloat16), w23[...],
                preferred_element_type=jnp.float32) + b23[...])
    dec = (jnp.dot(h2.astype(jnp.bfloat16), w4[...],
                   preferred_element_type=jnp.float32) + b4[...])
    d = x - dec
    if m_valid is not None:
        i = pl.program_id(0)
        rows = i * tile_m + jax.lax.broadcasted_iota(jnp.int32, (tile_m, 1), 0)
        d = jnp.where(rows < m_valid, d, 0.0)
    out_ref[0] = jnp.sum(d * d, axis=0, keepdims=True)


def _resident(arr):
    nd = arr.ndim
    return pl.BlockSpec(arr.shape, lambda *_: (0,) * nd)


TILE_M = 4096


def kernel(grid_feature, w1p, b1p, w2p, b2p, w3p, b3p, w4p, b4p):
    F = w1p.shape[0]
    x = jnp.asarray(grid_feature, jnp.float32).reshape(-1, F)
    m = x.shape[0]
    tile_m = min(TILE_M, _round_up(m, 8))
    m_pad = _round_up(m, tile_m)
    n_tiles = m_pad // tile_m
    if m_pad != m:
        x = jnp.pad(x, ((0, m_pad - m), (0, 0)))
    m_valid = None if m_pad == m else m

    w23 = w2p @ w3p
    b23 = b2p @ w3p + b3p

    MID = w1p.shape[1]
    weights = (w1p.astype(jnp.bfloat16), b1p,
               w23.astype(jnp.bfloat16), b23,
               w4p.astype(jnp.bfloat16), b4p)

    flops = 2 * m_pad * (F * MID + MID * MID + MID * F) + 4 * m_pad * F
    trans = 2 * m_pad * MID
    bytes_acc = 4 * m_pad * F + 2 * (2 * F * MID + MID * MID) + 4 * n_tiles * F

    partials = pl.pallas_call(
        functools.partial(_ae_loss_kernel, m_valid=m_valid, tile_m=tile_m),
        out_shape=jax.ShapeDtypeStruct((n_tiles, 1, F), jnp.float32),
        grid_spec=pltpu.PrefetchScalarGridSpec(
            num_scalar_prefetch=0,
            grid=(n_tiles,),
            in_specs=[pl.BlockSpec((tile_m, F), lambda i: (i, 0))]
                     + [_resident(w) for w in weights],
            out_specs=pl.BlockSpec((1, 1, F), lambda i: (i, 0, 0)),
        ),
        compiler_params=pltpu.CompilerParams(
            dimension_semantics=("parallel",),
            vmem_limit_bytes=64 * 1024 * 1024),
        cost_estimate=pl.CostEstimate(
            flops=flops, transcendentals=trans, bytes_accessed=bytes_acc),
    )(x, *weights)
    return jnp.sum(partials)


# speedup vs baseline: 1.5399x; 1.0365x over previous
import functools

import jax
import jax.numpy as jnp
from jax.experimental import pallas as pl
from jax.experimental.pallas import tpu as pltpu


def _round_up(x, m):
    return ((x + m - 1) // m) * m


def _ae_loss_kernel(x_ref, w1, b1, w23, b23, w4, b4, out_ref, *,
                    m_valid, tile_m):
    x = x_ref[...]
    xb = x.astype(jnp.bfloat16)
    h = jnp.tanh(
        jnp.dot(xb, w1[...], preferred_element_type=jnp.float32) + b1[...])
    h2 = jnp.tanh(
        jnp.dot(h.astype(jnp.bfloat16), w23[...],
                preferred_element_type=jnp.float32) + b23[...])
    dec = (jnp.dot(h2.astype(jnp.bfloat16), w4[...],
                   preferred_element_type=jnp.float32) + b4[...])
    d = x - dec
    if m_valid is not None:
        i = pl.program_id(0)
        rows = i * tile_m + jax.lax.broadcasted_iota(jnp.int32, (tile_m, 1), 0)
        d = jnp.where(rows < m_valid, d, 0.0)
    out_ref[0] = jnp.sum(d * d, axis=0, keepdims=True)


def _resident(arr):
    nd = arr.ndim
    return pl.BlockSpec(arr.shape, lambda *_: (0,) * nd)


TILE_M = 8192


def kernel(grid_feature, w1p, b1p, w2p, b2p, w3p, b3p, w4p, b4p):
    F = w1p.shape[0]
    x = jnp.asarray(grid_feature, jnp.float32).reshape(-1, F)
    m = x.shape[0]
    tile_m = min(TILE_M, _round_up(m, 8))
    m_pad = _round_up(m, tile_m)
    n_tiles = m_pad // tile_m
    if m_pad != m:
        x = jnp.pad(x, ((0, m_pad - m), (0, 0)))
    m_valid = None if m_pad == m else m

    w23 = w2p @ w3p
    b23 = b2p @ w3p + b3p

    MID = w1p.shape[1]
    weights = (w1p.astype(jnp.bfloat16), b1p,
               w23.astype(jnp.bfloat16), b23,
               w4p.astype(jnp.bfloat16), b4p)

    flops = 2 * m_pad * (F * MID + MID * MID + MID * F) + 4 * m_pad * F
    trans = 2 * m_pad * MID
    bytes_acc = 4 * m_pad * F + 2 * (2 * F * MID + MID * MID) + 4 * n_tiles * F

    partials = pl.pallas_call(
        functools.partial(_ae_loss_kernel, m_valid=m_valid, tile_m=tile_m),
        out_shape=jax.ShapeDtypeStruct((n_tiles, 1, F), jnp.float32),
        grid_spec=pltpu.PrefetchScalarGridSpec(
            num_scalar_prefetch=0,
            grid=(n_tiles,),
            in_specs=[pl.BlockSpec((tile_m, F), lambda i: (i, 0))]
                     + [_resident(w) for w in weights],
            out_specs=pl.BlockSpec((1, 1, F), lambda i: (i, 0, 0)),
        ),
        compiler_params=pltpu.CompilerParams(
            dimension_semantics=("parallel",),
            vmem_limit_bytes=64 * 1024 * 1024),
        cost_estimate=pl.CostEstimate(
            flops=flops, transcendentals=trans, bytes_accessed=bytes_acc),
    )(x, *weights)
    return jnp.sum(partials)
